# baseline (speedup 1.0000x reference)
import functools

import jax
import jax.numpy as jnp
from jax import lax
from jax.experimental import pallas as pl
from jax.experimental.pallas import tpu as pltpu
from jax.experimental.pallas import tpu_sc as plsc

VOCAB = 1000005
EMBED_DIM = 200
BATCH = 4096
SEQ_LEN = 50

NUM_CORES = 2
NUM_SUBCORES = 16
NUM_WORKERS = NUM_CORES * NUM_SUBCORES
B_TOTAL = BATCH * SEQ_LEN
ROWS_PER_W = B_TOTAL // NUM_WORKERS
CHUNK = 128
N_CHUNKS = ROWS_PER_W // CHUNK

_mesh = plsc.VectorSubcoreMesh(
    core_axis_name="c", subcore_axis_name="s",
    num_cores=NUM_CORES, num_subcores=NUM_SUBCORES,
)


@functools.partial(
    pl.kernel,
    out_type=jax.ShapeDtypeStruct((B_TOTAL, EMBED_DIM), jnp.float32),
    mesh=_mesh,
    scratch_types=[
        pltpu.VMEM((ROWS_PER_W,), jnp.int32),
        pltpu.VMEM((CHUNK, EMBED_DIM), jnp.float32),
        pltpu.SemaphoreType.DMA,
    ],
    compiler_params=pltpu.CompilerParams(use_tc_tiling_on_sc=False),
)
def _emb_lookup(idx_hbm, table_hbm, out_hbm, idx_v, rows_v, sem):
    wid = lax.axis_index("s") * NUM_CORES + lax.axis_index("c")
    base = wid * ROWS_PER_W
    pltpu.sync_copy(idx_hbm.at[pl.ds(base, ROWS_PER_W)], idx_v)

    def step(c, carry):
        off = pl.multiple_of(c * CHUNK, CHUNK)
        pltpu.async_copy(
            table_hbm.at[idx_v.at[pl.ds(off, CHUNK)]], rows_v, sem
        ).wait()
        pltpu.sync_copy(rows_v, out_hbm.at[pl.ds(base + off, CHUNK)])
        return carry

    lax.fori_loop(0, N_CHUNKS, step, 0)


def kernel(pad_indexes, embedding_table):
    idx = pad_indexes.reshape(-1)
    out = _emb_lookup(idx, embedding_table)
    return out.reshape(BATCH, SEQ_LEN, EMBED_DIM)


# speedup vs baseline: 1.0062x; 1.0062x over previous
import functools

import jax
import jax.numpy as jnp
from jax import lax
from jax.experimental import pallas as pl
from jax.experimental.pallas import tpu as pltpu
from jax.experimental.pallas import tpu_sc as plsc

VOCAB = 1000005
EMBED_DIM = 200
BATCH = 4096
SEQ_LEN = 50

NUM_CORES = 2
NUM_SUBCORES = 16
NUM_WORKERS = NUM_CORES * NUM_SUBCORES
B_TOTAL = BATCH * SEQ_LEN
ROWS_PER_W = B_TOTAL // NUM_WORKERS
CHUNK = 128
N_CHUNKS = ROWS_PER_W // CHUNK
NBUF = 4
N_ROUNDS = N_CHUNKS // NBUF

_mesh = plsc.VectorSubcoreMesh(
    core_axis_name="c", subcore_axis_name="s",
    num_cores=NUM_CORES, num_subcores=NUM_SUBCORES,
)


@functools.partial(
    pl.kernel,
    out_type=jax.ShapeDtypeStruct((B_TOTAL, EMBED_DIM), jnp.float32),
    mesh=_mesh,
    scratch_types=[
        pltpu.VMEM((ROWS_PER_W,), jnp.int32),
        [pltpu.VMEM((CHUNK, EMBED_DIM), jnp.float32) for _ in range(NBUF)],
        [pltpu.SemaphoreType.DMA for _ in range(NBUF)],
        [pltpu.SemaphoreType.DMA for _ in range(NBUF)],
    ],
    compiler_params=pltpu.CompilerParams(use_tc_tiling_on_sc=False),
)
def _emb_lookup(idx_hbm, table_hbm, out_hbm, idx_v, bufs, gsems, ssems):
    wid = lax.axis_index("s") * NUM_CORES + lax.axis_index("c")
    base = wid * ROWS_PER_W
    pltpu.sync_copy(idx_hbm.at[pl.ds(base, ROWS_PER_W)], idx_v)

    def gather(c, b):
        off = pl.multiple_of(c * CHUNK, CHUNK)
        return pltpu.async_copy(
            table_hbm.at[idx_v.at[pl.ds(off, CHUNK)]], bufs[b], gsems[b]
        )

    def wait_gather(c, b):
        off = pl.multiple_of(c * CHUNK, CHUNK)
        pltpu.make_async_copy(
            table_hbm.at[idx_v.at[pl.ds(off, CHUNK)]], bufs[b], gsems[b]
        ).wait()

    def scatter(c, b):
        off = pl.multiple_of(c * CHUNK, CHUNK)
        return pltpu.async_copy(
            bufs[b], out_hbm.at[pl.ds(base + off, CHUNK)], ssems[b]
        )

    for b in range(NBUF):
        gather(b, b)

    def round_body(r, carry):
        for b in range(NBUF):
            c = r * NBUF + b
            wait_gather(c, b)
            sd = scatter(c, b)
            nc = c + NBUF

            @pl.when(nc < N_CHUNKS)
            def _():
                sd.wait()
                gather(nc, b)

        return carry

    lax.fori_loop(0, N_ROUNDS, round_body, 0)

    for c in range(N_ROUNDS * NBUF, N_CHUNKS):
        b = c % NBUF
        wait_gather(c, b)
        scatter(c, b)
    for b in range(NBUF):
        pltpu.make_async_copy(
            bufs[b], out_hbm.at[pl.ds(base, CHUNK)], ssems[b]
        ).wait()


def kernel(pad_indexes, embedding_table):
    idx = pad_indexes.reshape(-1)
    out = _emb_lookup(idx, embedding_table)
    return out.reshape(BATCH, SEQ_LEN, EMBED_DIM)


# speedup vs baseline: 3.2311x; 3.2111x over previous
import functools

import jax
import jax.numpy as jnp
from jax import lax
from jax.experimental import pallas as pl
from jax.experimental.pallas import tpu as pltpu
from jax.experimental.pallas import tpu_sc as plsc

VOCAB = 1000005
EMBED_DIM = 200
BATCH = 4096
SEQ_LEN = 50

NUM_CORES = 2
NUM_SUBCORES = 16
NUM_WORKERS = NUM_CORES * NUM_SUBCORES
BATCH_PER_W = BATCH // NUM_WORKERS
N_ROUNDS = BATCH_PER_W // 2
NF = 10
_WINDOWS = (0, 16, 32, 34)

_mesh = plsc.VectorSubcoreMesh(
    core_axis_name="c", subcore_axis_name="s",
    num_cores=NUM_CORES, num_subcores=NUM_SUBCORES,
)


@functools.partial(
    pl.kernel,
    out_type=jax.ShapeDtypeStruct((BATCH, SEQ_LEN, EMBED_DIM), jnp.float32),
    mesh=_mesh,
    scratch_types=[
        pltpu.VMEM((BATCH_PER_W, SEQ_LEN), jnp.int32),
        pltpu.VMEM((NF, 8, EMBED_DIM), jnp.float32),
        [pltpu.VMEM((1, SEQ_LEN, EMBED_DIM), jnp.float32) for _ in range(2)],
        [pltpu.SemaphoreType.DMA for _ in range(NF)],
        [pltpu.SemaphoreType.DMA for _ in range(2)],
    ],
    compiler_params=pltpu.CompilerParams(use_tc_tiling_on_sc=True),
)
def _emb_lookup(idx_hbm, table_hbm, out_hbm, idx_v, fetch_v, stags, fsems, ssems):
    wid = lax.axis_index("s") * NUM_CORES + lax.axis_index("c")
    first_batch = pl.multiple_of(wid * BATCH_PER_W, 8)
    pltpu.sync_copy(idx_hbm.at[pl.ds(first_batch, BATCH_PER_W)], idx_v)

    def load_windows(q):
        return [idx_v[q, pl.ds(w, 16)] for w in _WINDOWS]

    def token_row(vecs, s):
        if s < 48:
            return vecs[s // 16][s % 16]
        return vecs[3][s - 34]

    def issue_fetch(row, slot):
        sub = lax.bitwise_and(row, 7)
        blk = pl.multiple_of(row - sub, 8)
        pltpu.async_copy(
            table_hbm.at[pl.ds(blk, 8)], fetch_v.at[slot], fsems[slot]
        )
        return sub

    def wait_fetch(slot):
        pltpu.make_async_copy(
            table_hbm.at[pl.ds(0, 8)], fetch_v.at[slot], fsems[slot]
        ).wait()

    def extract(slot, sub, stag, s):
        for col in tuple(range(0, EMBED_DIM - 16, 16)) + (EMBED_DIM - 16,):
            stag[0, s, pl.ds(col, 16)] = fetch_v.at[slot][sub, pl.ds(col, 16)]

    vecs0 = load_windows(0)
    for s in range(NF):
        issue_fetch(token_row(vecs0, s), s)

    def round_body(rnd, carry):
        for b in range(2):
            q = rnd * 2 + b
            stag = stags[b]

            @pl.when(rnd >= 1)
            def _():
                pltpu.make_async_copy(
                    stag, out_hbm.at[pl.ds(first_batch, 1)], ssems[b]
                ).wait()

            vecs = load_windows(q)
            for s in range(SEQ_LEN):
                slot = s % NF
                row = token_row(vecs, s)
                sub = lax.bitwise_and(row, 7)
                wait_fetch(slot)
                extract(slot, sub, stag, s)
                if s < SEQ_LEN - NF:
                    issue_fetch(token_row(vecs, s + NF), slot)
                elif b == 0:
                    vecs_n = load_windows(q + 1)
                    issue_fetch(token_row(vecs_n, s - (SEQ_LEN - NF)), slot)
                else:

                    @pl.when(q + 1 < BATCH_PER_W)
                    def _():
                        vecs_n = load_windows(q + 1)
                        issue_fetch(token_row(vecs_n, s - (SEQ_LEN - NF)), slot)

            pltpu.async_copy(
                stag, out_hbm.at[pl.ds(first_batch + q, 1)], ssems[b]
            )

        return carry

    lax.fori_loop(0, N_ROUNDS, round_body, 0)
    for b in range(2):
        pltpu.make_async_copy(
            stags[b], out_hbm.at[pl.ds(first_batch, 1)], ssems[b]
        ).wait()


def kernel(pad_indexes, embedding_table):
    return _emb_lookup(pad_indexes, embedding_table)
